# baseline (speedup 1.0000x reference)
import jax
import jax.numpy as jnp
from jax.experimental import pallas as pl
from jax.experimental.pallas import tpu as pltpu

N, D, NPROJ = 16384, 512, 1000
PPAD = 1024
PBLK = 128
TILE = 512


def _ce_small(x, j, take_min_if_bit_clear):
    r = x.shape[0]
    i = jax.lax.broadcasted_iota(jnp.int32, (r, 1), 0)
    bitj = (i & j) != 0
    up = jnp.concatenate([x[j:], x[:j]], axis=0)
    down = jnp.concatenate([x[-j:], x[:-j]], axis=0)
    if take_min_if_bit_clear:
        a = jnp.minimum(x, up)
        b = jnp.maximum(x, down)
    else:
        a = jnp.maximum(x, up)
        b = jnp.minimum(x, down)
    return jnp.where(bitj, b, a)


def _sort8(x, asc):
    for k, j in ((2, 1), (4, 2), (4, 1)):
        r = x.shape[0]
        i = jax.lax.broadcasted_iota(jnp.int32, (r, 1), 0)
        bitj = (i & j) != 0
        up = jnp.concatenate([x[j:], x[:j]], axis=0)
        down = jnp.concatenate([x[-j:], x[:-j]], axis=0)
        partner = jnp.where(bitj, down, up)
        mn = jnp.minimum(x, partner)
        mx = jnp.maximum(x, partner)
        take_min = ((i & k) == 0) != bitj
        x = jnp.where(take_min, mn, mx)
    for j in (4, 2, 1):
        x = _ce_small(x, j, asc)
    return x


def _merge_val(x, asc):
    r = x.shape[0]
    if r == 8:
        for j in (4, 2, 1):
            x = _ce_small(x, j, asc)
        return x
    h = r // 2
    a, b = x[:h], x[h:]
    mn = jnp.minimum(a, b)
    mx = jnp.maximum(a, b)
    lo, hi = (mn, mx) if asc else (mx, mn)
    return jnp.concatenate([_merge_val(lo, asc), _merge_val(hi, asc)], axis=0)


def _sort_val(x, asc):
    r = x.shape[0]
    if r == 8:
        return _sort8(x, asc)
    h = r // 2
    a = _sort_val(x[:h], True)
    b = _sort_val(x[h:], False)
    return _merge_val(jnp.concatenate([a, b], axis=0), asc)


_BATCHER8 = ((0, 1), (2, 3), (4, 5), (6, 7), (0, 2), (1, 3), (1, 2),
             (4, 6), (5, 7), (5, 6), (0, 4), (1, 5), (2, 6), (3, 7),
             (2, 4), (3, 5), (1, 2), (3, 4), (5, 6))


def _presort8(x):
    r, l = x.shape
    g = r // 8
    x3 = x.reshape(g, 8, l)
    p = [x3[:, q, :] for q in range(8)]
    for a, b in _BATCHER8:
        lo = jnp.minimum(p[a], p[b])
        hi = jnp.maximum(p[a], p[b])
        p[a], p[b] = lo, hi
    even = (jax.lax.broadcasted_iota(jnp.int32, (g, 1), 0) & 1) == 0
    cols = [jnp.where(even, p[q], p[7 - q])[:, None, :] for q in range(8)]
    return jnp.concatenate(cols, axis=1).reshape(r, l)


def _sort_val_presorted(x, asc):
    r = x.shape[0]
    if r == 8:
        return x
    h = r // 2
    a = _sort_val_presorted(x[:h], True)
    b = _sort_val_presorted(x[h:], False)
    return _merge_val(jnp.concatenate([a, b], axis=0), asc)


def _make_fused_kernel(n, d, tile, pblk, nblk):
    tiles = n // tile
    chunks = (n // 2) // tile
    lt = tile.bit_length() - 1
    mmrows = n // (tiles // 2)

    nchunk = tiles // 2

    def _fused_kernel(s_ref, t_ref, p_ref, out_ref, mm_ref, ss_ref, st_ref,
                      sem_ref):
        i = pl.program_id(0)
        par = i % 2
        prev = (i + 1) % 2

        def _al(t):
            return pl.multiple_of(t, tile)

        p = p_ref[...]
        nrm2 = jnp.sum(p * p, axis=0, keepdims=True)
        inv = jax.lax.rsqrt(jnp.where(nrm2 > 0, nrm2, 1.0))
        pn = p * inv
        do_mm = i < nblk
        do_sort = i > 0

        def _start_chunk(q, slot):
            rb = _al(q * mmrows)
            pltpu.make_async_copy(
                s_ref.at[pl.ds(rb, mmrows), :], ss_ref.at[slot],
                sem_ref.at[slot, 0]).start()
            pltpu.make_async_copy(
                t_ref.at[pl.ds(rb, mmrows), :], st_ref.at[slot],
                sem_ref.at[slot, 1]).start()

        def _wait_chunk(q, slot):
            rb = _al(q * mmrows)
            pltpu.make_async_copy(
                s_ref.at[pl.ds(rb, mmrows), :], ss_ref.at[slot],
                sem_ref.at[slot, 0]).wait()
            pltpu.make_async_copy(
                t_ref.at[pl.ds(rb, mmrows), :], st_ref.at[slot],
                sem_ref.at[slot, 1]).wait()

        @pl.when(do_mm)
        def _():
            _start_chunk(0, 0)

        def phase_a(q, carry):
            slot = q % 2

            @pl.when(do_mm & (q + 1 < nchunk))
            def _():
                _start_chunk(q + 1, (q + 1) % 2)

            @pl.when(do_mm)
            def _():
                _wait_chunk(q, slot)
                rb = _al(q * mmrows)
                mm_ref[par, pl.ds(rb, mmrows), 0:pblk] = jnp.dot(
                    ss_ref[slot], pn, preferred_element_type=jnp.float32)
                mm_ref[par, pl.ds(rb, mmrows), pblk:2 * pblk] = jnp.dot(
                    st_ref[slot], pn, preferred_element_type=jnp.float32)

            @pl.when(do_sort)
            def _():
                for parity, asc in ((0, True), (1, False)):
                    m = q * 2 + parity
                    base = _al(m * tile)
                    mm_ref[prev, pl.ds(base, tile), :] = _sort_val_presorted(
                        _presort8(mm_ref[prev, pl.ds(base, tile), :]), asc)
            return carry
        jax.lax.fori_loop(0, nchunk, phase_a, 0)

        @pl.when(do_sort)
        def _sort_rest():
            k = 2 * tile
            while k <= n:
                j = k // 2
                while j > tile:
                    lj = j.bit_length() - 1
                    csz = k >> (lt + 1)

                    def body(c, asc, lj=lj, j=j):
                        cb = c << lt
                        q = cb >> lj
                        rr = cb & (j - 1)
                        ia = _al((q << (lj + 1)) + rr)
                        ib = _al(ia + j)
                        a = mm_ref[prev, pl.ds(ia, tile), :]
                        b = mm_ref[prev, pl.ds(ib, tile), :]
                        mn = jnp.minimum(a, b)
                        mx = jnp.maximum(a, b)
                        lo, hi = (mn, mx) if asc else (mx, mn)
                        mm_ref[prev, pl.ds(ia, tile), :] = lo
                        mm_ref[prev, pl.ds(ib, tile), :] = hi

                    if csz >= chunks:
                        def phase_bg_all(c, carry, body=body):
                            body(c, True)
                            return carry
                        jax.lax.fori_loop(0, chunks, phase_bg_all, 0)
                    else:
                        def phase_bg(pp, carry, body=body, csz=csz):
                            c_asc = ((pp // csz) * 2) * csz + pp % csz
                            body(c_asc, True)
                            body(c_asc + csz, False)
                            return carry
                        jax.lax.fori_loop(0, chunks // 2, phase_bg, 0)
                    j //= 2

                csz = k >> (lt + 1)

                def fused(c, asc):
                    ia = _al(c * (2 * tile))
                    ib = _al(ia + tile)
                    a = mm_ref[prev, pl.ds(ia, tile), :]
                    b = mm_ref[prev, pl.ds(ib, tile), :]
                    mn = jnp.minimum(a, b)
                    mx = jnp.maximum(a, b)
                    lo, hi = (mn, mx) if asc else (mx, mn)
                    mm_ref[prev, pl.ds(ia, tile), :] = _merge_val(lo, asc)
                    mm_ref[prev, pl.ds(ib, tile), :] = _merge_val(hi, asc)

                if csz >= chunks:
                    def fused_all(c, carry):
                        fused(c, True)
                        return carry
                    jax.lax.fori_loop(0, chunks, fused_all, 0)
                else:
                    def fused_pair(pp, carry, csz=csz):
                        c_asc = ((pp // csz) * 2) * csz + pp % csz
                        fused(c_asc, True)
                        fused(c_asc + csz, False)
                        return carry
                    jax.lax.fori_loop(0, chunks // 2, fused_pair, 0)
                k *= 2

            def reduce_tile(m, acc):
                x = mm_ref[prev, pl.ds(_al(m * tile), tile), :]
                dd = x[:, :pblk] - x[:, pblk:]
                return acc + jnp.sum(dd * dd, axis=0, keepdims=True)
            acc = jax.lax.fori_loop(
                0, tiles, reduce_tile, jnp.zeros((1, pblk), jnp.float32))
            out_ref[...] = acc[None]

    return _fused_kernel


def _swd_partials(source, target, proj_padded, n=N, d=D, tile=TILE,
                  pblk=PBLK):
    ppad = proj_padded.shape[1]
    nblk = ppad // pblk
    grid = (nblk + 1,)
    return pl.pallas_call(
        _make_fused_kernel(n, d, tile, pblk, nblk),
        grid=grid,
        in_specs=[
            pl.BlockSpec(memory_space=pltpu.MemorySpace.HBM),
            pl.BlockSpec(memory_space=pltpu.MemorySpace.HBM),
            pl.BlockSpec((d, pblk), lambda i: (0, jnp.minimum(i, nblk - 1))),
        ],
        out_specs=pl.BlockSpec(
            (1, 1, pblk), lambda i: (jnp.maximum(i - 1, 0), 0, 0)),
        out_shape=jax.ShapeDtypeStruct((nblk, 1, pblk), jnp.float32),
        scratch_shapes=[
            pltpu.VMEM((2, n, 2 * pblk), jnp.float32),
            pltpu.VMEM((2, 2 * tile, d), jnp.float32),
            pltpu.VMEM((2, 2 * tile, d), jnp.float32),
            pltpu.SemaphoreType.DMA((2, 2)),
        ],
    )(source, target, proj_padded)


def kernel(source, target, proj):
    proj_padded = jnp.pad(proj, ((0, 0), (0, PPAD - NPROJ)))
    partial = _swd_partials(source, target, proj_padded)
    return jnp.sqrt(jnp.sum(partial) / (N * NPROJ))


# speedup vs baseline: 1.2593x; 1.2593x over previous
import jax
import jax.numpy as jnp
from jax.experimental import pallas as pl
from jax.experimental.pallas import tpu as pltpu

N, D, NPROJ = 16384, 512, 1000
PPAD = 1024
PBLK = 128
TILE = 512


def _ce_small(x, j, take_min_if_bit_clear):
    r = x.shape[0]
    i = jax.lax.broadcasted_iota(jnp.int32, (r, 1), 0)
    bitj = (i & j) != 0
    up = jnp.concatenate([x[j:], x[:j]], axis=0)
    down = jnp.concatenate([x[-j:], x[:-j]], axis=0)
    if take_min_if_bit_clear:
        a = jnp.minimum(x, up)
        b = jnp.maximum(x, down)
    else:
        a = jnp.maximum(x, up)
        b = jnp.minimum(x, down)
    return jnp.where(bitj, b, a)


def _sort8(x, asc):
    for k, j in ((2, 1), (4, 2), (4, 1)):
        r = x.shape[0]
        i = jax.lax.broadcasted_iota(jnp.int32, (r, 1), 0)
        bitj = (i & j) != 0
        up = jnp.concatenate([x[j:], x[:j]], axis=0)
        down = jnp.concatenate([x[-j:], x[:-j]], axis=0)
        partner = jnp.where(bitj, down, up)
        mn = jnp.minimum(x, partner)
        mx = jnp.maximum(x, partner)
        take_min = ((i & k) == 0) != bitj
        x = jnp.where(take_min, mn, mx)
    for j in (4, 2, 1):
        x = _ce_small(x, j, asc)
    return x


def _merge_val(x, asc):
    r = x.shape[0]
    if r == 8:
        for j in (4, 2, 1):
            x = _ce_small(x, j, asc)
        return x
    h = r // 2
    a, b = x[:h], x[h:]
    mn = jnp.minimum(a, b)
    mx = jnp.maximum(a, b)
    lo, hi = (mn, mx) if asc else (mx, mn)
    return jnp.concatenate([_merge_val(lo, asc), _merge_val(hi, asc)], axis=0)


def _sort_val(x, asc):
    r = x.shape[0]
    if r == 8:
        return _sort8(x, asc)
    h = r // 2
    a = _sort_val(x[:h], True)
    b = _sort_val(x[h:], False)
    return _merge_val(jnp.concatenate([a, b], axis=0), asc)


def _make_fused_kernel(n, d, tile, pblk, nblk):
    tiles = n // tile
    chunks = (n // 2) // tile
    lt = tile.bit_length() - 1
    mmrows = n // (tiles // 2)

    nchunk = tiles // 2

    def _fused_kernel(s_ref, t_ref, p_ref, out_ref, mm_ref, ss_ref, st_ref,
                      sem_ref):
        i = pl.program_id(0)
        par = i % 2
        prev = (i + 1) % 2

        def _al(t):
            return pl.multiple_of(t, tile)

        p = p_ref[...]
        nrm2 = jnp.sum(p * p, axis=0, keepdims=True)
        inv = jax.lax.rsqrt(jnp.where(nrm2 > 0, nrm2, 1.0))
        pn = p * inv
        do_mm = i < nblk
        do_sort = i > 0

        def _start_chunk(q, slot):
            rb = _al(q * mmrows)
            pltpu.make_async_copy(
                s_ref.at[pl.ds(rb, mmrows), :], ss_ref.at[slot],
                sem_ref.at[slot, 0]).start()
            pltpu.make_async_copy(
                t_ref.at[pl.ds(rb, mmrows), :], st_ref.at[slot],
                sem_ref.at[slot, 1]).start()

        def _wait_chunk(q, slot):
            rb = _al(q * mmrows)
            pltpu.make_async_copy(
                s_ref.at[pl.ds(rb, mmrows), :], ss_ref.at[slot],
                sem_ref.at[slot, 0]).wait()
            pltpu.make_async_copy(
                t_ref.at[pl.ds(rb, mmrows), :], st_ref.at[slot],
                sem_ref.at[slot, 1]).wait()

        @pl.when(do_mm)
        def _():
            _start_chunk(0, 0)

        def phase_a(q, carry):
            slot = q % 2

            @pl.when(do_mm & (q + 1 < nchunk))
            def _():
                _start_chunk(q + 1, (q + 1) % 2)

            @pl.when(do_mm)
            def _():
                _wait_chunk(q, slot)
                rb = _al(q * mmrows)
                mm_ref[par, pl.ds(rb, mmrows), 0:pblk] = jnp.dot(
                    ss_ref[slot], pn, preferred_element_type=jnp.float32)
                mm_ref[par, pl.ds(rb, mmrows), pblk:2 * pblk] = jnp.dot(
                    st_ref[slot], pn, preferred_element_type=jnp.float32)

            @pl.when(do_sort)
            def _():
                for parity, asc in ((0, True), (1, False)):
                    m = q * 2 + parity
                    base = _al(m * tile)
                    mm_ref[prev, pl.ds(base, tile), :] = _sort_val(
                        mm_ref[prev, pl.ds(base, tile), :], asc)
            return carry
        jax.lax.fori_loop(0, nchunk, phase_a, 0)

        @pl.when(do_sort)
        def _sort_rest():
            k = 2 * tile
            while k <= n:
                j = k // 2
                while j > tile:
                    lj = j.bit_length() - 1
                    csz = k >> (lt + 1)

                    def body(c, asc, lj=lj, j=j):
                        cb = c << lt
                        q = cb >> lj
                        rr = cb & (j - 1)
                        ia = _al((q << (lj + 1)) + rr)
                        ib = _al(ia + j)
                        a = mm_ref[prev, pl.ds(ia, tile), :]
                        b = mm_ref[prev, pl.ds(ib, tile), :]
                        mn = jnp.minimum(a, b)
                        mx = jnp.maximum(a, b)
                        lo, hi = (mn, mx) if asc else (mx, mn)
                        mm_ref[prev, pl.ds(ia, tile), :] = lo
                        mm_ref[prev, pl.ds(ib, tile), :] = hi

                    if csz >= chunks:
                        def phase_bg_all(c, carry, body=body):
                            body(c, True)
                            return carry
                        jax.lax.fori_loop(0, chunks, phase_bg_all, 0)
                    else:
                        def phase_bg(pp, carry, body=body, csz=csz):
                            c_asc = ((pp // csz) * 2) * csz + pp % csz
                            body(c_asc, True)
                            body(c_asc + csz, False)
                            return carry
                        jax.lax.fori_loop(0, chunks // 2, phase_bg, 0)
                    j //= 2

                csz = k >> (lt + 1)

                def fused(c, asc):
                    ia = _al(c * (2 * tile))
                    ib = _al(ia + tile)
                    a = mm_ref[prev, pl.ds(ia, tile), :]
                    b = mm_ref[prev, pl.ds(ib, tile), :]
                    mn = jnp.minimum(a, b)
                    mx = jnp.maximum(a, b)
                    lo, hi = (mn, mx) if asc else (mx, mn)
                    mm_ref[prev, pl.ds(ia, tile), :] = _merge_val(lo, asc)
                    mm_ref[prev, pl.ds(ib, tile), :] = _merge_val(hi, asc)

                if csz >= chunks:
                    def fused_all(c, carry):
                        fused(c, True)
                        return carry
                    jax.lax.fori_loop(0, chunks, fused_all, 0)
                else:
                    def fused_pair(pp, carry, csz=csz):
                        c_asc = ((pp // csz) * 2) * csz + pp % csz
                        fused(c_asc, True)
                        fused(c_asc + csz, False)
                        return carry
                    jax.lax.fori_loop(0, chunks // 2, fused_pair, 0)
                k *= 2

            def reduce_tile(m, acc):
                x = mm_ref[prev, pl.ds(_al(m * tile), tile), :]
                dd = x[:, :pblk] - x[:, pblk:]
                return acc + jnp.sum(dd * dd, axis=0, keepdims=True)
            acc = jax.lax.fori_loop(
                0, tiles, reduce_tile, jnp.zeros((1, pblk), jnp.float32))
            out_ref[...] = acc[None]

    return _fused_kernel


def _swd_partials(source, target, proj_padded, n=N, d=D, tile=TILE,
                  pblk=PBLK):
    ppad = proj_padded.shape[1]
    nblk = ppad // pblk
    grid = (nblk + 1,)
    return pl.pallas_call(
        _make_fused_kernel(n, d, tile, pblk, nblk),
        grid=grid,
        in_specs=[
            pl.BlockSpec(memory_space=pltpu.MemorySpace.HBM),
            pl.BlockSpec(memory_space=pltpu.MemorySpace.HBM),
            pl.BlockSpec((d, pblk), lambda i: (0, jnp.minimum(i, nblk - 1))),
        ],
        out_specs=pl.BlockSpec(
            (1, 1, pblk), lambda i: (jnp.maximum(i - 1, 0), 0, 0)),
        out_shape=jax.ShapeDtypeStruct((nblk, 1, pblk), jnp.float32),
        scratch_shapes=[
            pltpu.VMEM((2, n, 2 * pblk), jnp.float32),
            pltpu.VMEM((2, 2 * tile, d), jnp.float32),
            pltpu.VMEM((2, 2 * tile, d), jnp.float32),
            pltpu.SemaphoreType.DMA((2, 2)),
        ],
    )(source, target, proj_padded)


def kernel(source, target, proj):
    proj_padded = jnp.pad(proj, ((0, 0), (0, PPAD - NPROJ)))
    partial = _swd_partials(source, target, proj_padded)
    return jnp.sqrt(jnp.sum(partial) / (N * NPROJ))
